# baseline (speedup 1.0000x reference)
import functools

import jax
import jax.numpy as jnp
from jax import lax
from jax.experimental import pallas as pl
from jax.experimental.pallas import tpu as pltpu
from jax.experimental.pallas import tpu_sc as plsc

N_NODES = 10000
N_PAD = 10240
D = 128
N_EDGES = 320000

NC = 2
NS = 16
NW = NC * NS
E_PER_W = N_EDGES // NW
CHUNK = 80
N_CHUNKS = E_PER_W // CHUNK
ROWS_PER_TILE = N_PAD // NS


def _zero_vmem_2d(ref, rows, cols):
    def row_body(r, _):
        for j in range(cols // 16):
            ref[r, pl.ds(j * 16, 16)] = jnp.zeros((16,), jnp.float32)
        return 0

    lax.fori_loop(0, rows, row_body, 0)


def _zero_vmem_1d(ref, n):
    def body(k, _):
        ref[pl.ds(k * 16, 16)] = jnp.zeros((16,), jnp.float32)
        return 0

    lax.fori_loop(0, n // 16, body, 0)


def _make_seg_sum(compute_deg: bool):
    mesh = plsc.VectorSubcoreMesh(core_axis_name="c", subcore_axis_name="s")

    out_type = [jax.ShapeDtypeStruct((N_PAD, D), jnp.float32) for _ in range(NC)]
    if compute_deg:
        out_type += [jax.ShapeDtypeStruct((N_PAD,), jnp.float32) for _ in range(NC)]

    scratch = [
        pltpu.VMEM((N_CHUNKS, CHUNK), jnp.int32),
        pltpu.VMEM((CHUNK,), jnp.int32),
        pltpu.VMEM((CHUNK,), jnp.int32),
        pltpu.VMEM((CHUNK,), jnp.int32),
        pltpu.VMEM((CHUNK,), jnp.int32),
        pltpu.VMEM((CHUNK, D), jnp.float32),
        pltpu.VMEM((CHUNK, D), jnp.float32),
        pltpu.VMEM((CHUNK,), jnp.float32),
        pltpu.VMEM((256,), jnp.float32),
        pltpu.VMEM_SHARED((N_PAD, D), jnp.float32),
        pltpu.VMEM_SHARED((N_PAD,), jnp.float32),
        pltpu.SemaphoreType.DMA,
        pltpu.SemaphoreType.DMA,
        pltpu.SemaphoreType.DMA,
        pltpu.SemaphoreType.DMA,
    ]

    def body(x_hbm, pk_hbm, *rest):
        p_hbm = rest[:NC]
        d_hbm = rest[NC:2 * NC] if compute_deg else None
        scr = rest[2 * NC:] if compute_deg else rest[NC:]
        (pk, sx_a, sx_b, dx_a, dx_b, rows_a, rows_b, ones_v, zdeg, acc_sh,
         deg_sh, gsem_a, gsem_b, ssem_a, ssem_b) = scr
        sx = (sx_a, sx_b)
        dx = (dx_a, dx_b)
        rows = (rows_a, rows_b)
        gsem = (gsem_a, gsem_b)
        ssem = (ssem_a, ssem_b)

        cid = lax.axis_index("c")
        sid = lax.axis_index("s")
        w = cid * NS + sid
        row0 = sid * ROWS_PER_TILE

        pltpu.async_copy(pk_hbm.at[w], pk, gsem_b)

        _zero_vmem_2d(rows_a, CHUNK, D)
        for k in range(ROWS_PER_TILE // CHUNK):
            pltpu.sync_copy(rows_a, acc_sh.at[pl.ds(row0 + k * CHUNK, CHUNK)])
        if compute_deg:
            _zero_vmem_1d(zdeg, 256)
            for j in range(CHUNK // 16):
                ones_v[pl.ds(j * 16, 16)] = jnp.ones((16,), jnp.float32)

            @pl.when(sid < N_PAD // 1024)
            def _():
                for j in range(4):
                    pltpu.sync_copy(
                        zdeg, deg_sh.at[pl.ds(sid * 1024 + j * 256, 256)])

        pltpu.make_async_copy(pk_hbm.at[w], pk, gsem_b).wait()
        plsc.subcore_barrier()

        def unpack(i, b):
            for j in range(CHUNK // 16):
                v = pk[i, pl.ds(j * 16, 16)]
                sx[b][pl.ds(j * 16, 16)] = v & 0xFFFF
                dx[b][pl.ds(j * 16, 16)] = v >> 16

        def gath(b):
            pltpu.async_copy(x_hbm.at[sx[b]], rows[b], gsem[b])

        def gwait(b):
            pltpu.make_async_copy(x_hbm.at[sx[b]], rows[b], gsem[b]).wait()

        def scat(b):
            pltpu.async_copy(rows[b], acc_sh.at[dx[b]], ssem[b], add=True)
            if compute_deg:
                pltpu.async_copy(ones_v, deg_sh.at[dx[b]], ssem[b], add=True)

        def swait(b):
            pltpu.make_async_copy(rows[b], acc_sh.at[dx[b]], ssem[b]).wait()
            if compute_deg:
                pltpu.make_async_copy(ones_v, deg_sh.at[dx[b]], ssem[b]).wait()

        unpack(0, 0)
        gath(0)
        gwait(0)
        scat(0)
        unpack(1, 1)
        gath(1)

        @pl.loop(0, (N_CHUNKS - 3) // 2)
        def _(k):
            i1 = 2 * k + 1
            swait(0)
            unpack(i1 + 1, 0)
            gath(0)
            gwait(1)
            scat(1)
            swait(1)
            unpack(i1 + 2, 1)
            gath(1)
            gwait(0)
            scat(0)

        swait(0)
        unpack(N_CHUNKS - 1, 0)
        gath(0)
        gwait(1)
        scat(1)
        swait(1)
        gwait(0)
        scat(0)
        swait(0)

        plsc.subcore_barrier()

        for c in range(NC):
            @pl.when(cid == c)
            def _(c=c):
                pltpu.sync_copy(acc_sh.at[pl.ds(row0, ROWS_PER_TILE)],
                                p_hbm[c].at[pl.ds(row0, ROWS_PER_TILE)])
                if compute_deg:
                    @pl.when(sid < N_PAD // 1024)
                    def _():
                        pltpu.sync_copy(deg_sh.at[pl.ds(sid * 1024, 1024)],
                                        d_hbm[c].at[pl.ds(sid * 1024, 1024)])

    return pl.kernel(body, out_type=out_type, mesh=mesh, scratch_types=scratch)


_seg_sum_deg = _make_seg_sum(compute_deg=True)
_seg_sum = _make_seg_sum(compute_deg=False)


def _tc_pre_body(x_ref, wr_ref, b_ref, o_ref):
    o_ref[...] = (jnp.dot(x_ref[...], wr_ref[...],
                          preferred_element_type=jnp.float32) + b_ref[...])


def _tc_pre(x, wr, b):
    R = 1000
    return pl.pallas_call(
        _tc_pre_body,
        grid=(N_NODES // R,),
        in_specs=[
            pl.BlockSpec((R, D), lambda i: (i, 0)),
            pl.BlockSpec((D, D), lambda i: (0, 0)),
            pl.BlockSpec((1, D), lambda i: (0, 0)),
        ],
        out_specs=pl.BlockSpec((R, D), lambda i: (i, 0)),
        out_shape=jax.ShapeDtypeStruct((N_NODES, D), jnp.float32),
    )(x, wr, b)


def _tc_post_body(relu, p0_ref, p1_ref, d0_ref, d1_ref, xr_ref, wl_ref,
                  o_ref):
    s = p0_ref[...] + p1_ref[...]
    deg = jnp.maximum(d0_ref[...] + d1_ref[...], 1.0)
    mean = s / deg
    acc = (jnp.dot(mean, wl_ref[...], preferred_element_type=jnp.float32)
           + xr_ref[...])
    o_ref[...] = jnp.maximum(acc, 0.0) if relu else acc


def _tc_post(p0, p1, d0, d1, xr, wl, relu):
    R = 1000
    return pl.pallas_call(
        functools.partial(_tc_post_body, relu),
        grid=(N_NODES // R,),
        in_specs=[
            pl.BlockSpec((R, D), lambda i: (i, 0)),
            pl.BlockSpec((R, D), lambda i: (i, 0)),
            pl.BlockSpec((R, 1), lambda i: (i, 0)),
            pl.BlockSpec((R, 1), lambda i: (i, 0)),
            pl.BlockSpec((R, D), lambda i: (i, 0)),
            pl.BlockSpec((D, D), lambda i: (0, 0)),
        ],
        out_specs=pl.BlockSpec((R, D), lambda i: (i, 0)),
        out_shape=jax.ShapeDtypeStruct((N_NODES, D), jnp.float32),
    )(p0, p1, d0, d1, xr, wl)


def kernel(x, edge_index, Wl1, Wr1, b1, Wl2, Wr2, b2):
    src = edge_index[0].astype(jnp.int32)
    dst = edge_index[1].astype(jnp.int32)
    pk = (src | (dst << 16)).reshape(NW, N_CHUNKS, CHUNK)

    xr1 = _tc_pre(x, Wr1, b1.reshape(1, D))
    p0, p1, dg0, dg1 = _seg_sum_deg(x, pk)
    d0 = dg0.reshape(N_PAD, 1)
    d1 = dg1.reshape(N_PAD, 1)
    h = _tc_post(p0, p1, d0, d1, xr1, Wl1, relu=True)
    xr2 = _tc_pre(h, Wr2, b2.reshape(1, D))
    q0, q1 = _seg_sum(h, pk)
    out = _tc_post(q0, q1, d0, d1, xr2, Wl2, relu=False)
    return out


# speedup vs baseline: 1.0164x; 1.0164x over previous
import functools

import jax
import jax.numpy as jnp
from jax import lax
from jax.experimental import pallas as pl
from jax.experimental.pallas import tpu as pltpu
from jax.experimental.pallas import tpu_sc as plsc

N_NODES = 10000
N_PAD = 10240
D = 128
N_EDGES = 320000

NC = 2
NS = 16
NW = NC * NS
E_PER_W = N_EDGES // NW
CHUNK = 80
N_CHUNKS = E_PER_W // CHUNK
ROWS_PER_TILE = N_PAD // NS


def _zero_vmem_2d(ref, rows, cols):
    def row_body(r, _):
        for j in range(cols // 16):
            ref[r, pl.ds(j * 16, 16)] = jnp.zeros((16,), jnp.float32)
        return 0

    lax.fori_loop(0, rows, row_body, 0)


def _zero_vmem_1d(ref, n):
    def body(k, _):
        ref[pl.ds(k * 16, 16)] = jnp.zeros((16,), jnp.float32)
        return 0

    lax.fori_loop(0, n // 16, body, 0)


def _make_seg_sum(compute_deg: bool):
    mesh = plsc.VectorSubcoreMesh(core_axis_name="c", subcore_axis_name="s")

    out_type = [jax.ShapeDtypeStruct((N_PAD, D), jnp.float32) for _ in range(NC)]
    if compute_deg:
        out_type += [jax.ShapeDtypeStruct((N_PAD,), jnp.float32) for _ in range(NC)]

    scratch = [
        pltpu.VMEM((N_CHUNKS, CHUNK), jnp.int32),
        pltpu.VMEM((CHUNK,), jnp.int32),
        pltpu.VMEM((CHUNK,), jnp.int32),
        pltpu.VMEM((CHUNK,), jnp.int32),
        pltpu.VMEM((CHUNK,), jnp.int32),
        pltpu.VMEM((CHUNK, D), jnp.float32),
        pltpu.VMEM((CHUNK, D), jnp.float32),
        pltpu.VMEM((CHUNK,), jnp.float32),
        pltpu.VMEM((256,), jnp.float32),
        pltpu.VMEM_SHARED((N_PAD, D), jnp.float32),
        pltpu.VMEM_SHARED((N_PAD,), jnp.float32),
        pltpu.SemaphoreType.DMA,
        pltpu.SemaphoreType.DMA,
        pltpu.SemaphoreType.DMA,
        pltpu.SemaphoreType.DMA,
    ]

    def body(x_hbm, pk_hbm, *rest):
        p_hbm = rest[:NC]
        d_hbm = rest[NC:2 * NC] if compute_deg else None
        scr = rest[2 * NC:] if compute_deg else rest[NC:]
        (pk, sx_a, sx_b, dx_a, dx_b, rows_a, rows_b, ones_v, zdeg, acc_sh,
         deg_sh, gsem_a, gsem_b, ssem_a, ssem_b) = scr
        sx = (sx_a, sx_b)
        dx = (dx_a, dx_b)
        rows = (rows_a, rows_b)
        gsem = (gsem_a, gsem_b)
        ssem = (ssem_a, ssem_b)

        cid = lax.axis_index("c")
        sid = lax.axis_index("s")
        w = cid * NS + sid
        row0 = sid * ROWS_PER_TILE

        pltpu.async_copy(pk_hbm.at[w], pk, gsem_b)

        _zero_vmem_2d(rows_a, CHUNK, D)
        for k in range(ROWS_PER_TILE // CHUNK):
            pltpu.sync_copy(rows_a, acc_sh.at[pl.ds(row0 + k * CHUNK, CHUNK)])
        if compute_deg:
            _zero_vmem_1d(zdeg, 256)
            for j in range(CHUNK // 16):
                ones_v[pl.ds(j * 16, 16)] = jnp.ones((16,), jnp.float32)

            @pl.when(sid < N_PAD // 1024)
            def _():
                for j in range(4):
                    pltpu.sync_copy(
                        zdeg, deg_sh.at[pl.ds(sid * 1024 + j * 256, 256)])

        pltpu.make_async_copy(pk_hbm.at[w], pk, gsem_b).wait()
        plsc.subcore_barrier()

        def unpack(i, b):
            for j in range(CHUNK // 16):
                v = pk[i, pl.ds(j * 16, 16)]
                sx[b][pl.ds(j * 16, 16)] = v & 0xFFFF
                dx[b][pl.ds(j * 16, 16)] = v >> 16

        def gath(b):
            pltpu.async_copy(x_hbm.at[sx[b]], rows[b], gsem[b])

        def gwait(b):
            pltpu.make_async_copy(x_hbm.at[sx[b]], rows[b], gsem[b]).wait()

        def scat(b):
            pltpu.async_copy(rows[b], acc_sh.at[dx[b]], ssem[b], add=True)
            if compute_deg:
                pltpu.async_copy(ones_v, deg_sh.at[dx[b]], ssem[b], add=True)

        def swait(b):
            pltpu.make_async_copy(rows[b], acc_sh.at[dx[b]], ssem[b]).wait()
            if compute_deg:
                pltpu.make_async_copy(ones_v, deg_sh.at[dx[b]], ssem[b]).wait()

        unpack(0, 0)
        gath(0)
        gwait(0)
        scat(0)
        unpack(1, 1)
        gath(1)

        @pl.loop(0, (N_CHUNKS - 3) // 2)
        def _(k):
            i1 = 2 * k + 1
            swait(0)
            unpack(i1 + 1, 0)
            gath(0)
            gwait(1)
            scat(1)
            swait(1)
            unpack(i1 + 2, 1)
            gath(1)
            gwait(0)
            scat(0)

        swait(0)
        unpack(N_CHUNKS - 1, 0)
        gath(0)
        gwait(1)
        scat(1)
        swait(1)
        gwait(0)
        scat(0)
        swait(0)

        plsc.subcore_barrier()

        for c in range(NC):
            @pl.when(cid == c)
            def _(c=c):
                pltpu.sync_copy(acc_sh.at[pl.ds(row0, ROWS_PER_TILE)],
                                p_hbm[c].at[pl.ds(row0, ROWS_PER_TILE)])
                if compute_deg:
                    @pl.when(sid < N_PAD // 1024)
                    def _():
                        pltpu.sync_copy(deg_sh.at[pl.ds(sid * 1024, 1024)],
                                        d_hbm[c].at[pl.ds(sid * 1024, 1024)])

    return pl.kernel(body, out_type=out_type, mesh=mesh, scratch_types=scratch)


_seg_sum_deg = _make_seg_sum(compute_deg=True)
_seg_sum = _make_seg_sum(compute_deg=False)


def _pack_body(e_ref, o_ref):
    o_ref[...] = e_ref[0] | (e_ref[1] << 16)


def _pack_edges(edge_index):
    e = edge_index.astype(jnp.int32).reshape(2, N_EDGES // D, D)
    pk = pl.pallas_call(
        _pack_body,
        out_shape=jax.ShapeDtypeStruct((N_EDGES // D, D), jnp.int32),
    )(e)
    return pk.reshape(NW, N_CHUNKS, CHUNK)


def _tc_pre_body(x_ref, wr_ref, b_ref, o_ref):
    o_ref[...] = (jnp.dot(x_ref[...], wr_ref[...],
                          preferred_element_type=jnp.float32) + b_ref[...])


def _tc_pre(x, wr, b):
    R = 1000
    return pl.pallas_call(
        _tc_pre_body,
        grid=(N_NODES // R,),
        in_specs=[
            pl.BlockSpec((R, D), lambda i: (i, 0)),
            pl.BlockSpec((D, D), lambda i: (0, 0)),
            pl.BlockSpec((1, D), lambda i: (0, 0)),
        ],
        out_specs=pl.BlockSpec((R, D), lambda i: (i, 0)),
        out_shape=jax.ShapeDtypeStruct((N_NODES, D), jnp.float32),
    )(x, wr, b)


def _tc_post_body(relu, p0_ref, p1_ref, d0_ref, d1_ref, xr_ref, wl_ref,
                  o_ref):
    s = p0_ref[...] + p1_ref[...]
    deg = jnp.maximum(d0_ref[...] + d1_ref[...], 1.0)
    mean = s / deg
    acc = (jnp.dot(mean, wl_ref[...], preferred_element_type=jnp.float32)
           + xr_ref[...])
    o_ref[...] = jnp.maximum(acc, 0.0) if relu else acc


def _tc_post(p0, p1, d0, d1, xr, wl, relu):
    R = 1000
    return pl.pallas_call(
        functools.partial(_tc_post_body, relu),
        grid=(N_NODES // R,),
        in_specs=[
            pl.BlockSpec((R, D), lambda i: (i, 0)),
            pl.BlockSpec((R, D), lambda i: (i, 0)),
            pl.BlockSpec((R, 1), lambda i: (i, 0)),
            pl.BlockSpec((R, 1), lambda i: (i, 0)),
            pl.BlockSpec((R, D), lambda i: (i, 0)),
            pl.BlockSpec((D, D), lambda i: (0, 0)),
        ],
        out_specs=pl.BlockSpec((R, D), lambda i: (i, 0)),
        out_shape=jax.ShapeDtypeStruct((N_NODES, D), jnp.float32),
    )(p0, p1, d0, d1, xr, wl)


def kernel(x, edge_index, Wl1, Wr1, b1, Wl2, Wr2, b2):
    pk = _pack_edges(edge_index)
    xr1 = _tc_pre(x, Wr1, b1.reshape(1, D))
    p0, p1, dg0, dg1 = _seg_sum_deg(x, pk)
    d0 = dg0.reshape(N_PAD, 1)
    d1 = dg1.reshape(N_PAD, 1)
    h = _tc_post(p0, p1, d0, d1, xr1, Wl1, relu=True)
    xr2 = _tc_pre(h, Wr2, b2.reshape(1, D))
    q0, q1 = _seg_sum(h, pk)
    out = _tc_post(q0, q1, d0, d1, xr2, Wl2, relu=False)
    return out


# speedup vs baseline: 1.0364x; 1.0197x over previous
import functools

import jax
import jax.numpy as jnp
from jax import lax
from jax.experimental import pallas as pl
from jax.experimental.pallas import tpu as pltpu
from jax.experimental.pallas import tpu_sc as plsc

N_NODES = 10000
N_PAD = 10240
D = 128
N_EDGES = 320000

NC = 2
NS = 16
NW = NC * NS
E_PER_W = N_EDGES // NW
CHUNK = 80
N_CHUNKS = E_PER_W // CHUNK
ROWS_PER_TILE = N_PAD // NS


def _zero_vmem_2d(ref, rows, cols):
    def row_body(r, _):
        for j in range(cols // 16):
            ref[r, pl.ds(j * 16, 16)] = jnp.zeros((16,), jnp.float32)
        return 0

    lax.fori_loop(0, rows, row_body, 0)


def _zero_vmem_1d(ref, n):
    def body(k, _):
        ref[pl.ds(k * 16, 16)] = jnp.zeros((16,), jnp.float32)
        return 0

    lax.fori_loop(0, n // 16, body, 0)


def _make_seg_sum(compute_deg: bool):
    mesh = plsc.VectorSubcoreMesh(core_axis_name="c", subcore_axis_name="s")

    out_type = [jax.ShapeDtypeStruct((N_PAD, D), jnp.float32) for _ in range(NC)]
    if compute_deg:
        out_type += [jax.ShapeDtypeStruct((N_PAD,), jnp.float32) for _ in range(NC)]

    scratch = [
        pltpu.VMEM((N_CHUNKS, CHUNK), jnp.int32),
        pltpu.VMEM((CHUNK,), jnp.int32),
        pltpu.VMEM((CHUNK,), jnp.int32),
        pltpu.VMEM((CHUNK,), jnp.int32),
        pltpu.VMEM((CHUNK,), jnp.int32),
        pltpu.VMEM((CHUNK, D), jnp.float32),
        pltpu.VMEM((CHUNK, D), jnp.float32),
        pltpu.VMEM((CHUNK,), jnp.float32),
        pltpu.VMEM((256,), jnp.float32),
        pltpu.VMEM_SHARED((N_PAD, D), jnp.float32),
        pltpu.VMEM_SHARED((N_PAD,), jnp.float32),
        pltpu.SemaphoreType.DMA,
        pltpu.SemaphoreType.DMA,
        pltpu.SemaphoreType.DMA,
        pltpu.SemaphoreType.DMA,
    ]

    def body(x_hbm, pk_hbm, *rest):
        p_hbm = rest[:NC]
        d_hbm = rest[NC:2 * NC] if compute_deg else None
        scr = rest[2 * NC:] if compute_deg else rest[NC:]
        (pk, sx_a, sx_b, dx_a, dx_b, rows_a, rows_b, ones_v, zdeg, acc_sh,
         deg_sh, gsem_a, gsem_b, ssem_a, ssem_b) = scr
        sx = (sx_a, sx_b)
        dx = (dx_a, dx_b)
        rows = (rows_a, rows_b)
        gsem = (gsem_a, gsem_b)
        ssem = (ssem_a, ssem_b)

        cid = lax.axis_index("c")
        sid = lax.axis_index("s")
        w = cid * NS + sid
        row0 = sid * ROWS_PER_TILE

        pltpu.async_copy(pk_hbm.at[w], pk, gsem_b)

        _zero_vmem_2d(rows_a, CHUNK, D)
        for k in range(ROWS_PER_TILE // CHUNK):
            pltpu.sync_copy(rows_a, acc_sh.at[pl.ds(row0 + k * CHUNK, CHUNK)])
        if compute_deg:
            _zero_vmem_1d(zdeg, 256)
            for j in range(CHUNK // 16):
                ones_v[pl.ds(j * 16, 16)] = jnp.ones((16,), jnp.float32)

            @pl.when(sid < N_PAD // 1024)
            def _():
                for j in range(4):
                    pltpu.sync_copy(
                        zdeg, deg_sh.at[pl.ds(sid * 1024 + j * 256, 256)])

        pltpu.make_async_copy(pk_hbm.at[w], pk, gsem_b).wait()
        plsc.subcore_barrier()

        def unpack(i, b):
            for j in range(CHUNK // 16):
                v = pk[i, pl.ds(j * 16, 16)]
                sx[b][pl.ds(j * 16, 16)] = v & 0xFFFF
                dx[b][pl.ds(j * 16, 16)] = v >> 16

        def gath(b):
            pltpu.async_copy(x_hbm.at[sx[b]], rows[b], gsem[b])

        def gwait(b):
            pltpu.make_async_copy(x_hbm.at[sx[b]], rows[b], gsem[b]).wait()

        def scat(b):
            pltpu.async_copy(rows[b], acc_sh.at[dx[b]], ssem[b], add=True)
            if compute_deg:
                pltpu.async_copy(ones_v, deg_sh.at[dx[b]], ssem[b], add=True)

        def swait(b):
            pltpu.make_async_copy(rows[b], acc_sh.at[dx[b]], ssem[b]).wait()
            if compute_deg:
                pltpu.make_async_copy(ones_v, deg_sh.at[dx[b]], ssem[b]).wait()

        unpack(0, 0)
        gath(0)
        gwait(0)
        scat(0)
        unpack(1, 1)
        gath(1)

        @pl.loop(0, (N_CHUNKS - 3) // 2)
        def _(k):
            i1 = 2 * k + 1
            swait(0)
            unpack(i1 + 1, 0)
            gath(0)
            gwait(1)
            scat(1)
            swait(1)
            unpack(i1 + 2, 1)
            gath(1)
            gwait(0)
            scat(0)

        swait(0)
        unpack(N_CHUNKS - 1, 0)
        gath(0)
        gwait(1)
        scat(1)
        swait(1)
        gwait(0)
        scat(0)
        swait(0)

        plsc.subcore_barrier()

        for c in range(NC):
            @pl.when(cid == c)
            def _(c=c):
                pltpu.sync_copy(acc_sh.at[pl.ds(row0, ROWS_PER_TILE)],
                                p_hbm[c].at[pl.ds(row0, ROWS_PER_TILE)])
                if compute_deg:
                    @pl.when(sid < N_PAD // 1024)
                    def _():
                        pltpu.sync_copy(deg_sh.at[pl.ds(sid * 1024, 1024)],
                                        d_hbm[c].at[pl.ds(sid * 1024, 1024)])

    return pl.kernel(body, out_type=out_type, mesh=mesh, scratch_types=scratch)


_seg_sum_deg = _make_seg_sum(compute_deg=True)
_seg_sum = _make_seg_sum(compute_deg=False)


def _pack_body(e_ref, o_ref):
    o_ref[...] = e_ref[0] | (e_ref[1] << 16)


def _pack_edges(edge_index):
    e = edge_index.astype(jnp.int32).reshape(2, N_EDGES // D, D)
    pk = pl.pallas_call(
        _pack_body,
        out_shape=jax.ShapeDtypeStruct((N_EDGES // D, D), jnp.int32),
    )(e)
    return pk.reshape(NW, N_CHUNKS, CHUNK)


def _tc_pre_body(x_ref, wr_ref, b_ref, o_ref):
    o_ref[...] = (jnp.dot(x_ref[...], wr_ref[...],
                          preferred_element_type=jnp.float32) + b_ref[...])


def _tc_pre(x, wr, b):
    R = 2000
    return pl.pallas_call(
        _tc_pre_body,
        grid=(N_NODES // R,),
        in_specs=[
            pl.BlockSpec((R, D), lambda i: (i, 0)),
            pl.BlockSpec((D, D), lambda i: (0, 0)),
            pl.BlockSpec((1, D), lambda i: (0, 0)),
        ],
        out_specs=pl.BlockSpec((R, D), lambda i: (i, 0)),
        out_shape=jax.ShapeDtypeStruct((N_NODES, D), jnp.float32),
    )(x, wr, b)


def _tc_post_body(relu, p0_ref, p1_ref, d0_ref, d1_ref, xr_ref, wl_ref,
                  o_ref):
    s = p0_ref[...] + p1_ref[...]
    deg = jnp.maximum(d0_ref[...] + d1_ref[...], 1.0)
    mean = s / deg
    acc = (jnp.dot(mean, wl_ref[...], preferred_element_type=jnp.float32)
           + xr_ref[...])
    o_ref[...] = jnp.maximum(acc, 0.0) if relu else acc


def _tc_post(p0, p1, d0, d1, xr, wl, relu):
    R = 2000
    return pl.pallas_call(
        functools.partial(_tc_post_body, relu),
        grid=(N_NODES // R,),
        in_specs=[
            pl.BlockSpec((R, D), lambda i: (i, 0)),
            pl.BlockSpec((R, D), lambda i: (i, 0)),
            pl.BlockSpec((R, 1), lambda i: (i, 0)),
            pl.BlockSpec((R, 1), lambda i: (i, 0)),
            pl.BlockSpec((R, D), lambda i: (i, 0)),
            pl.BlockSpec((D, D), lambda i: (0, 0)),
        ],
        out_specs=pl.BlockSpec((R, D), lambda i: (i, 0)),
        out_shape=jax.ShapeDtypeStruct((N_NODES, D), jnp.float32),
    )(p0, p1, d0, d1, xr, wl)


def kernel(x, edge_index, Wl1, Wr1, b1, Wl2, Wr2, b2):
    pk = _pack_edges(edge_index)
    xr1 = _tc_pre(x, Wr1, b1.reshape(1, D))
    p0, p1, dg0, dg1 = _seg_sum_deg(x, pk)
    d0 = dg0.reshape(N_PAD, 1)
    d1 = dg1.reshape(N_PAD, 1)
    h = _tc_post(p0, p1, d0, d1, xr1, Wl1, relu=True)
    xr2 = _tc_pre(h, Wr2, b2.reshape(1, D))
    q0, q1 = _seg_sum(h, pk)
    out = _tc_post(q0, q1, d0, d1, xr2, Wl2, relu=False)
    return out


# speedup vs baseline: 1.0707x; 1.0331x over previous
import functools

import jax
import jax.numpy as jnp
from jax import lax
from jax.experimental import pallas as pl
from jax.experimental.pallas import tpu as pltpu
from jax.experimental.pallas import tpu_sc as plsc

N_NODES = 10000
N_PAD = 10240
D = 128
N_EDGES = 320000

NC = 2
NS = 16
NW = NC * NS
E_PER_W = N_EDGES // NW
CHUNK = 80
N_CHUNKS = E_PER_W // CHUNK
ROWS_PER_TILE = N_PAD // NS


def _zero_vmem_2d(ref, rows, cols):
    def row_body(r, _):
        for j in range(cols // 16):
            ref[r, pl.ds(j * 16, 16)] = jnp.zeros((16,), jnp.float32)
        return 0

    lax.fori_loop(0, rows, row_body, 0)


def _zero_vmem_1d(ref, n):
    def body(k, _):
        ref[pl.ds(k * 16, 16)] = jnp.zeros((16,), jnp.float32)
        return 0

    lax.fori_loop(0, n // 16, body, 0)


def _make_seg_sum(compute_deg: bool):
    mesh = plsc.VectorSubcoreMesh(core_axis_name="c", subcore_axis_name="s")

    out_type = [jax.ShapeDtypeStruct((N_PAD, D), jnp.float32) for _ in range(NC)]
    if compute_deg:
        out_type += [jax.ShapeDtypeStruct((N_PAD,), jnp.float32) for _ in range(NC)]

    scratch = [
        pltpu.VMEM((E_PER_W,), jnp.int32),
        pltpu.VMEM((CHUNK,), jnp.int32),
        pltpu.VMEM((CHUNK,), jnp.int32),
        pltpu.VMEM((CHUNK,), jnp.int32),
        pltpu.VMEM((CHUNK,), jnp.int32),
        pltpu.VMEM((CHUNK, D), jnp.float32),
        pltpu.VMEM((CHUNK, D), jnp.float32),
        pltpu.VMEM((CHUNK,), jnp.float32),
        pltpu.VMEM((256,), jnp.float32),
        pltpu.VMEM_SHARED((N_PAD, D), jnp.float32),
        pltpu.VMEM_SHARED((N_PAD,), jnp.float32),
        pltpu.SemaphoreType.DMA,
        pltpu.SemaphoreType.DMA,
        pltpu.SemaphoreType.DMA,
        pltpu.SemaphoreType.DMA,
    ]

    def body(x_hbm, pk_hbm, *rest):
        p_hbm = rest[:NC]
        d_hbm = rest[NC:2 * NC] if compute_deg else None
        scr = rest[2 * NC:] if compute_deg else rest[NC:]
        (pk, sx_a, sx_b, dx_a, dx_b, rows_a, rows_b, ones_v, zdeg, acc_sh,
         deg_sh, gsem_a, gsem_b, ssem_a, ssem_b) = scr
        sx = (sx_a, sx_b)
        dx = (dx_a, dx_b)
        rows = (rows_a, rows_b)
        gsem = (gsem_a, gsem_b)
        ssem = (ssem_a, ssem_b)

        cid = lax.axis_index("c")
        sid = lax.axis_index("s")
        w = cid * NS + sid
        row0 = sid * ROWS_PER_TILE

        pltpu.async_copy(pk_hbm.at[pl.ds(w * E_PER_W, E_PER_W)], pk, gsem_b)

        _zero_vmem_2d(rows_a, CHUNK, D)
        for k in range(ROWS_PER_TILE // CHUNK):
            pltpu.sync_copy(rows_a, acc_sh.at[pl.ds(row0 + k * CHUNK, CHUNK)])
        if compute_deg:
            _zero_vmem_1d(zdeg, 256)
            for j in range(CHUNK // 16):
                ones_v[pl.ds(j * 16, 16)] = jnp.ones((16,), jnp.float32)

            @pl.when(sid < N_PAD // 1024)
            def _():
                for j in range(4):
                    pltpu.sync_copy(
                        zdeg, deg_sh.at[pl.ds(sid * 1024 + j * 256, 256)])

        pltpu.make_async_copy(
            pk_hbm.at[pl.ds(w * E_PER_W, E_PER_W)], pk, gsem_b).wait()
        plsc.subcore_barrier()

        def unpack(i, b):
            for j in range(CHUNK // 16):
                v = pk[pl.ds(i * CHUNK + j * 16, 16)]
                sx[b][pl.ds(j * 16, 16)] = v & 0xFFFF
                dx[b][pl.ds(j * 16, 16)] = v >> 16

        def gath(b):
            pltpu.async_copy(x_hbm.at[sx[b]], rows[b], gsem[b])

        def gwait(b):
            pltpu.make_async_copy(x_hbm.at[sx[b]], rows[b], gsem[b]).wait()

        def scat(b):
            pltpu.async_copy(rows[b], acc_sh.at[dx[b]], ssem[b], add=True)
            if compute_deg:
                pltpu.async_copy(ones_v, deg_sh.at[dx[b]], ssem[b], add=True)

        def swait(b):
            pltpu.make_async_copy(rows[b], acc_sh.at[dx[b]], ssem[b]).wait()
            if compute_deg:
                pltpu.make_async_copy(ones_v, deg_sh.at[dx[b]], ssem[b]).wait()

        unpack(0, 0)
        gath(0)
        gwait(0)
        scat(0)
        unpack(1, 1)
        gath(1)

        @pl.loop(0, (N_CHUNKS - 3) // 2)
        def _(k):
            i1 = 2 * k + 1
            swait(0)
            unpack(i1 + 1, 0)
            gath(0)
            gwait(1)
            scat(1)
            swait(1)
            unpack(i1 + 2, 1)
            gath(1)
            gwait(0)
            scat(0)

        swait(0)
        unpack(N_CHUNKS - 1, 0)
        gath(0)
        gwait(1)
        scat(1)
        swait(1)
        gwait(0)
        scat(0)
        swait(0)

        plsc.subcore_barrier()

        for c in range(NC):
            @pl.when(cid == c)
            def _(c=c):
                pltpu.sync_copy(acc_sh.at[pl.ds(row0, ROWS_PER_TILE)],
                                p_hbm[c].at[pl.ds(row0, ROWS_PER_TILE)])
                if compute_deg:
                    @pl.when(sid < N_PAD // 1024)
                    def _():
                        pltpu.sync_copy(deg_sh.at[pl.ds(sid * 1024, 1024)],
                                        d_hbm[c].at[pl.ds(sid * 1024, 1024)])

    return pl.kernel(body, out_type=out_type, mesh=mesh, scratch_types=scratch)


_seg_sum_deg = _make_seg_sum(compute_deg=True)
_seg_sum = _make_seg_sum(compute_deg=False)


def _pack_body(e_ref, o_ref):
    o_ref[...] = e_ref[0] | (e_ref[1] << 16)


def _pack_edges(edge_index):
    e = edge_index.astype(jnp.int32)
    return pl.pallas_call(
        _pack_body,
        out_shape=jax.ShapeDtypeStruct((N_EDGES,), jnp.int32),
    )(e)


def _tc_pre_body(x_ref, wr_ref, b_ref, o_ref):
    o_ref[...] = (jnp.dot(x_ref[...], wr_ref[...],
                          preferred_element_type=jnp.float32) + b_ref[...])


def _tc_pre(x, wr, b):
    R = 2000
    return pl.pallas_call(
        _tc_pre_body,
        grid=(N_NODES // R,),
        in_specs=[
            pl.BlockSpec((R, D), lambda i: (i, 0)),
            pl.BlockSpec((D, D), lambda i: (0, 0)),
            pl.BlockSpec((1, D), lambda i: (0, 0)),
        ],
        out_specs=pl.BlockSpec((R, D), lambda i: (i, 0)),
        out_shape=jax.ShapeDtypeStruct((N_NODES, D), jnp.float32),
    )(x, wr, b)


def _tc_post_body(relu, p0_ref, p1_ref, d0_ref, d1_ref, xr_ref, wl_ref,
                  o_ref):
    s = p0_ref[...] + p1_ref[...]
    deg = jnp.maximum(d0_ref[...] + d1_ref[...], 1.0)
    mean = s / deg
    acc = (jnp.dot(mean, wl_ref[...], preferred_element_type=jnp.float32)
           + xr_ref[...])
    o_ref[...] = jnp.maximum(acc, 0.0) if relu else acc


def _tc_post(p0, p1, d0, d1, xr, wl, relu):
    R = 2000
    return pl.pallas_call(
        functools.partial(_tc_post_body, relu),
        grid=(N_NODES // R,),
        in_specs=[
            pl.BlockSpec((R, D), lambda i: (i, 0)),
            pl.BlockSpec((R, D), lambda i: (i, 0)),
            pl.BlockSpec((R, 1), lambda i: (i, 0)),
            pl.BlockSpec((R, 1), lambda i: (i, 0)),
            pl.BlockSpec((R, D), lambda i: (i, 0)),
            pl.BlockSpec((D, D), lambda i: (0, 0)),
        ],
        out_specs=pl.BlockSpec((R, D), lambda i: (i, 0)),
        out_shape=jax.ShapeDtypeStruct((N_NODES, D), jnp.float32),
    )(p0, p1, d0, d1, xr, wl)


def kernel(x, edge_index, Wl1, Wr1, b1, Wl2, Wr2, b2):
    pk = _pack_edges(edge_index)
    xr1 = _tc_pre(x, Wr1, b1.reshape(1, D))
    p0, p1, dg0, dg1 = _seg_sum_deg(x, pk)
    d0 = dg0.reshape(N_PAD, 1)
    d1 = dg1.reshape(N_PAD, 1)
    h = _tc_post(p0, p1, d0, d1, xr1, Wl1, relu=True)
    xr2 = _tc_pre(h, Wr2, b2.reshape(1, D))
    q0, q1 = _seg_sum(h, pk)
    out = _tc_post(q0, q1, d0, d1, xr2, Wl2, relu=False)
    return out
